# baseline (speedup 1.0000x reference)
import jax
import jax.numpy as jnp
from jax.experimental import pallas as pl
from jax.experimental.pallas import tpu as pltpu

_CHUNK = 2048


def _gather_body(zrow_ref, table_ref, out_ref):
    zrow = zrow_ref[...].reshape(1, -1)
    tv = table_ref[...]
    v = tv.shape[0]
    onehot_t = (zrow == jax.lax.broadcasted_iota(
        jnp.int32, (v, zrow.shape[1]), 0)).astype(jnp.bfloat16)
    t_hi = tv.astype(jnp.bfloat16)
    t_lo = (tv - t_hi.astype(jnp.float32)).astype(jnp.bfloat16)
    t_cat = jnp.concatenate([t_hi, t_lo], axis=1)
    dims = (((0,), (0,)), ((), ()))
    res = jax.lax.dot_general(onehot_t, t_cat, dimension_numbers=dims,
                              preferred_element_type=jnp.float32)
    d = tv.shape[1]
    out_ref[...] = res[:, :d] + res[:, d:]


def _tc_gather(table, idx):
    B = idx.shape[0]
    V, D = table.shape
    grid = ((B + _CHUNK - 1) // _CHUNK,)
    return pl.pallas_call(
        _gather_body,
        grid=grid,
        in_specs=[
            pl.BlockSpec((_CHUNK,), lambda i: (i,)),
            pl.BlockSpec((V, D), lambda i: (0, 0)),
        ],
        compiler_params=pltpu.CompilerParams(
            fuse_transposed_lhs_in_matmul=True),
        out_specs=pl.BlockSpec((_CHUNK, D), lambda i: (i, 0)),
        out_shape=jax.ShapeDtypeStruct((B, D), jnp.float32),
    )(idx, table)


def kernel(z, graph, edges_dist, orientation, table):
    del orientation
    zi = z.astype(jnp.int32)
    node_scalars = _tc_gather(table, zi)
    node_vectors = jnp.zeros((graph.shape[0], 3, table.shape[1]),
                             dtype=edges_dist.dtype)
    return (node_scalars, node_vectors)


# speedup vs baseline: 1.0065x; 1.0065x over previous
import jax
import jax.numpy as jnp
from jax.experimental import pallas as pl
from jax.experimental.pallas import tpu as pltpu

_CHUNK = 4096


def _gather_body(zrow_ref, table_ref, out_ref):
    zrow = zrow_ref[...].reshape(1, -1)
    tv = table_ref[...]
    v = tv.shape[0]
    onehot_t = (zrow == jax.lax.broadcasted_iota(
        jnp.int32, (v, zrow.shape[1]), 0)).astype(jnp.bfloat16)
    t_hi = tv.astype(jnp.bfloat16)
    t_lo = (tv - t_hi.astype(jnp.float32)).astype(jnp.bfloat16)
    t_cat = jnp.concatenate([t_hi, t_lo], axis=1)
    dims = (((0,), (0,)), ((), ()))
    res = jax.lax.dot_general(onehot_t, t_cat, dimension_numbers=dims,
                              preferred_element_type=jnp.float32)
    d = tv.shape[1]
    out_ref[...] = res[:, :d] + res[:, d:]


def _tc_gather(table, idx):
    B = idx.shape[0]
    V, D = table.shape
    grid = ((B + _CHUNK - 1) // _CHUNK,)
    return pl.pallas_call(
        _gather_body,
        grid=grid,
        in_specs=[
            pl.BlockSpec((_CHUNK,), lambda i: (i,)),
            pl.BlockSpec((V, D), lambda i: (0, 0)),
        ],
        compiler_params=pltpu.CompilerParams(
            fuse_transposed_lhs_in_matmul=True),
        out_specs=pl.BlockSpec((_CHUNK, D), lambda i: (i, 0)),
        out_shape=jax.ShapeDtypeStruct((B, D), jnp.float32),
    )(idx, table)


def kernel(z, graph, edges_dist, orientation, table):
    del orientation
    zi = z.astype(jnp.int32)
    node_scalars = _tc_gather(table, zi)
    node_vectors = jnp.zeros((graph.shape[0], 3, table.shape[1]),
                             dtype=edges_dist.dtype)
    return (node_scalars, node_vectors)


# speedup vs baseline: 1.0086x; 1.0021x over previous
import jax
import jax.numpy as jnp
from jax.experimental import pallas as pl
from jax.experimental.pallas import tpu as pltpu

_CHUNK = 5120


def _gather_body(zrow_ref, table_ref, out_ref):
    zrow = zrow_ref[...].reshape(1, -1)
    tv = table_ref[...]
    v = tv.shape[0]
    onehot_t = (zrow == jax.lax.broadcasted_iota(
        jnp.int32, (v, zrow.shape[1]), 0)).astype(jnp.bfloat16)
    t_hi = tv.astype(jnp.bfloat16)
    t_lo = (tv - t_hi.astype(jnp.float32)).astype(jnp.bfloat16)
    t_cat = jnp.concatenate([t_hi, t_lo], axis=1)
    dims = (((0,), (0,)), ((), ()))
    res = jax.lax.dot_general(onehot_t, t_cat, dimension_numbers=dims,
                              preferred_element_type=jnp.float32)
    d = tv.shape[1]
    out_ref[...] = res[:, :d] + res[:, d:]


def _tc_gather(table, idx):
    B = idx.shape[0]
    V, D = table.shape
    grid = ((B + _CHUNK - 1) // _CHUNK,)
    return pl.pallas_call(
        _gather_body,
        grid=grid,
        in_specs=[
            pl.BlockSpec((_CHUNK,), lambda i: (i,)),
            pl.BlockSpec((V, D), lambda i: (0, 0)),
        ],
        compiler_params=pltpu.CompilerParams(
            fuse_transposed_lhs_in_matmul=True),
        out_specs=pl.BlockSpec((_CHUNK, D), lambda i: (i, 0)),
        out_shape=jax.ShapeDtypeStruct((B, D), jnp.float32),
    )(idx, table)


def kernel(z, graph, edges_dist, orientation, table):
    del orientation
    zi = z.astype(jnp.int32)
    node_scalars = _tc_gather(table, zi)
    node_vectors = jnp.zeros((graph.shape[0], 3, table.shape[1]),
                             dtype=edges_dist.dtype)
    return (node_scalars, node_vectors)
